# baseline (speedup 1.0000x reference)
import jax
import jax.numpy as jnp
from jax import lax
from jax.experimental import pallas as pl
from jax.experimental.pallas import tpu as pltpu

_EPS = 1e-5


def _round_up(x, m):
    return -(-x // m) * m


def _mlp3_body(x_ref, w_ref, b_ref, o_ref, *, dim_in, l, dim_out_p):
    w1 = w_ref[0:dim_in, :]
    w2 = w_ref[dim_in:dim_in + l, :]
    w3 = w_ref[dim_in + l:dim_in + 2 * l, 0:dim_out_p]
    t1 = b_ref[0:1, :]
    t2 = b_ref[1:2, :]
    b3 = b_ref[2:3, 0:dim_out_p]

    x = x_ref[...].astype(jnp.bfloat16)
    h = jnp.dot(x, w1, preferred_element_type=jnp.float32)
    h = jnp.maximum(h + t1, 0.0).astype(jnp.bfloat16)
    h = jnp.dot(h, w2, preferred_element_type=jnp.float32)
    h = jnp.maximum(h + t2, 0.0).astype(jnp.bfloat16)
    o_ref[...] = (jnp.dot(h, w3, preferred_element_type=jnp.float32) + b3)


def kernel(x, w1, b1, g1, be1, m1, v1, w2, b2, g2, be2, m2, v2, w3, b3):
    s1 = g1 * lax.rsqrt(v1 + _EPS)
    w1f = (w1 * s1).astype(jnp.bfloat16)
    b1f = (b1 - m1) * s1 + be1
    s2 = g2 * lax.rsqrt(v2 + _EPS)
    w2f = (w2 * s2).astype(jnp.bfloat16)
    b2f = (b2 - m2) * s2 + be2

    B, dim_in = x.shape
    l = w1f.shape[1]
    dim_out = w3.shape[1]
    dim_out_p = max(128, _round_up(dim_out, 128))

    w3p = jnp.pad(w3, ((0, 0), (0, l - dim_out))).astype(jnp.bfloat16)
    b3p = jnp.pad(b3, ((0, 0), (0, l - dim_out)))
    wcat = jnp.concatenate([w1f, w2f, w3p], axis=0)
    bcat = jnp.concatenate([b1f, b2f, b3p], axis=0)

    TB = 1024 if B >= 1024 else max(8, _round_up(B, 8))
    B_pad = _round_up(B, TB)
    if B_pad != B:
        x = jnp.pad(x, ((0, B_pad - B), (0, 0)))
    grid = (B_pad // TB,)

    bf2, f4 = 2, 4
    footprint = (bf2 * (dim_in + 2 * l) * l + f4 * 3 * l
                 + 2 * (f4 * TB * dim_in + f4 * TB * dim_out_p)
                 + f4 * TB * l + bf2 * TB * l)
    vmem_limit = int(min(max(2 * footprint, 16 << 20), 60 << 20))

    import functools
    body = functools.partial(_mlp3_body, dim_in=dim_in, l=l,
                             dim_out_p=dim_out_p)
    out_p = pl.pallas_call(
        body,
        out_shape=jax.ShapeDtypeStruct((B_pad, dim_out_p), jnp.float32),
        grid=grid,
        in_specs=[
            pl.BlockSpec((TB, dim_in), lambda i: (i, 0)),
            pl.BlockSpec(wcat.shape, lambda i: (0, 0)),
            pl.BlockSpec(bcat.shape, lambda i: (0, 0)),
        ],
        out_specs=pl.BlockSpec((TB, dim_out_p), lambda i: (i, 0)),
        compiler_params=pltpu.CompilerParams(
            dimension_semantics=("arbitrary",),
            vmem_limit_bytes=vmem_limit,
        ),
    )(x, wcat, bcat)

    return out_p[:B, :dim_out]


# speedup vs baseline: 1.2523x; 1.2523x over previous
import jax
import jax.numpy as jnp
from jax import lax
from jax.experimental import pallas as pl
from jax.experimental.pallas import tpu as pltpu

_EPS = 1e-5


def _round_up(x, m):
    return -(-x // m) * m


def _mlp3_body(x_ref, w1_ref, b1_ref, w2_ref, b2_ref, w3_ref, b3_ref, o_ref):
    x = x_ref[...].astype(jnp.bfloat16)
    h = jnp.dot(x, w1_ref[...], preferred_element_type=jnp.float32)
    h = jnp.maximum(h + b1_ref[...], 0.0).astype(jnp.bfloat16)
    h = jnp.dot(h, w2_ref[...], preferred_element_type=jnp.float32)
    h = jnp.maximum(h + b2_ref[...], 0.0).astype(jnp.bfloat16)
    o_ref[...] = (jnp.dot(h, w3_ref[...], preferred_element_type=jnp.float32)
                  + b3_ref[...]).astype(o_ref.dtype)


def kernel(x, w1, b1, g1, be1, m1, v1, w2, b2, g2, be2, m2, v2, w3, b3):
    s1 = g1 * lax.rsqrt(v1 + _EPS)
    w1f = (w1 * s1).astype(jnp.bfloat16)
    b1f = (b1 - m1) * s1 + be1
    s2 = g2 * lax.rsqrt(v2 + _EPS)
    w2f = (w2 * s2).astype(jnp.bfloat16)
    b2f = (b2 - m2) * s2 + be2

    B, dim_in = x.shape
    l = w1f.shape[1]
    dim_out = w3.shape[1]
    dim_out_p = max(128, _round_up(dim_out, 128))
    if dim_out_p != dim_out:
        w3 = jnp.pad(w3, ((0, 0), (0, dim_out_p - dim_out)))
        b3 = jnp.pad(b3, ((0, 0), (0, dim_out_p - dim_out)))
    w3b = w3.astype(jnp.bfloat16)

    TB = 1024 if B >= 1024 else max(8, _round_up(B, 8))
    B_pad = _round_up(B, TB)
    if B_pad != B:
        x = jnp.pad(x, ((0, B_pad - B), (0, 0)))
    grid = (B_pad // TB,)

    bf2, f4 = 2, 4
    footprint = (bf2 * (dim_in * l + l * l + l * dim_out_p)
                 + f4 * (2 * l + dim_out_p)
                 + 2 * (f4 * TB * dim_in + f4 * TB * dim_out_p)
                 + f4 * TB * l + bf2 * TB * l)
    vmem_limit = int(min(max(2 * footprint, 16 << 20), 60 << 20))

    const = lambda shape: pl.BlockSpec(shape, lambda i: (0, 0))
    out_p = pl.pallas_call(
        _mlp3_body,
        out_shape=jax.ShapeDtypeStruct((B_pad, dim_out_p), jnp.float32),
        grid=grid,
        in_specs=[
            pl.BlockSpec((TB, dim_in), lambda i: (i, 0)),
            const(w1f.shape), const(b1f.shape),
            const(w2f.shape), const(b2f.shape),
            const(w3b.shape), const(b3.shape),
        ],
        out_specs=pl.BlockSpec((TB, dim_out_p), lambda i: (i, 0)),
        compiler_params=pltpu.CompilerParams(
            dimension_semantics=("parallel",),
            vmem_limit_bytes=vmem_limit,
        ),
    )(x, w1f, b1f, w2f, b2f, w3b, b3)

    return out_p[:B, :dim_out]


# speedup vs baseline: 1.2579x; 1.0045x over previous
import jax
import jax.numpy as jnp
from jax import lax
from jax.experimental import pallas as pl
from jax.experimental.pallas import tpu as pltpu

_EPS = 1e-5


def _round_up(x, m):
    return -(-x // m) * m


def _mlp3_body(x_ref, w1_ref, b1_ref, w2_ref, b2_ref, w3_ref, b3_ref, o_ref,
               w1s, w2s, w3s):
    @pl.when(pl.program_id(0) == 0)
    def _copy():
        w1s[...] = w1_ref[...]
        w2s[...] = w2_ref[...]
        w3s[...] = w3_ref[...]

    x = x_ref[...].astype(jnp.bfloat16)
    h = jnp.dot(x, w1s[...], preferred_element_type=jnp.float32)
    h = jnp.maximum(h + b1_ref[...], 0.0).astype(jnp.bfloat16)
    h = jnp.dot(h, w2s[...], preferred_element_type=jnp.float32)
    h = jnp.maximum(h + b2_ref[...], 0.0).astype(jnp.bfloat16)
    o_ref[...] = (jnp.dot(h, w3s[...], preferred_element_type=jnp.float32)
                  + b3_ref[...]).astype(o_ref.dtype)


def kernel(x, w1, b1, g1, be1, m1, v1, w2, b2, g2, be2, m2, v2, w3, b3):
    s1 = g1 * lax.rsqrt(v1 + _EPS)
    w1f = (w1 * s1).astype(jnp.bfloat16)
    b1f = (b1 - m1) * s1 + be1
    s2 = g2 * lax.rsqrt(v2 + _EPS)
    w2f = (w2 * s2).astype(jnp.bfloat16)
    b2f = (b2 - m2) * s2 + be2

    B, dim_in = x.shape
    l = w1f.shape[1]
    dim_out = w3.shape[1]
    dim_out_p = max(128, _round_up(dim_out, 128))
    if dim_out_p != dim_out:
        w3 = jnp.pad(w3, ((0, 0), (0, dim_out_p - dim_out)))
        b3 = jnp.pad(b3, ((0, 0), (0, dim_out_p - dim_out)))
    w3b = w3.astype(jnp.bfloat16)

    TB = 1024 if B >= 1024 else max(8, _round_up(B, 8))
    B_pad = _round_up(B, TB)
    if B_pad != B:
        x = jnp.pad(x, ((0, B_pad - B), (0, 0)))
    grid = (B_pad // TB,)

    bf2, f4 = 2, 4
    footprint = (bf2 * (dim_in * l + l * l + l * dim_out_p)
                 + f4 * (2 * l + dim_out_p)
                 + 2 * (f4 * TB * dim_in + f4 * TB * dim_out_p)
                 + f4 * TB * l + bf2 * TB * l)
    vmem_limit = int(min(max(2 * footprint, 16 << 20), 60 << 20))

    const = lambda shape: pl.BlockSpec(shape, lambda i: (0, 0))
    out_p = pl.pallas_call(
        _mlp3_body,
        out_shape=jax.ShapeDtypeStruct((B_pad, dim_out_p), jnp.float32),
        grid=grid,
        in_specs=[
            pl.BlockSpec((TB, dim_in), lambda i: (i, 0)),
            const(w1f.shape), const(b1f.shape),
            const(w2f.shape), const(b2f.shape),
            const(w3b.shape), const(b3.shape),
        ],
        out_specs=pl.BlockSpec((TB, dim_out_p), lambda i: (i, 0)),
        scratch_shapes=[
            pltpu.VMEM(w1f.shape, jnp.bfloat16),
            pltpu.VMEM(w2f.shape, jnp.bfloat16),
            pltpu.VMEM(w3b.shape, jnp.bfloat16),
        ],
        compiler_params=pltpu.CompilerParams(
            dimension_semantics=("parallel",),
            vmem_limit_bytes=vmem_limit,
        ),
    )(x, w1f, b1f, w2f, b2f, w3b, b3)

    return out_p[:B, :dim_out]


# speedup vs baseline: 1.2654x; 1.0060x over previous
import functools

import jax
import jax.numpy as jnp
from jax import lax
from jax.experimental import pallas as pl
from jax.experimental.pallas import tpu as pltpu

_EPS = 1e-5


def _round_up(x, m):
    return -(-x // m) * m


def _mlp3_body(x_ref, w1_ref, w2_ref, w3_ref, v_ref, o_ref,
               w1s, w2s, w3s, *, dim_out_p):
    @pl.when(pl.program_id(0) == 0)
    def _fold():
        w1s[...] = (w1_ref[...] * v_ref[0:1, :]).astype(jnp.bfloat16)
        w2s[...] = (w2_ref[...] * v_ref[2:3, :]).astype(jnp.bfloat16)
        w3s[...] = w3_ref[...].astype(jnp.bfloat16)

    t1 = v_ref[1:2, :]
    t2 = v_ref[3:4, :]
    b3 = v_ref[4:5, 0:dim_out_p]

    x = x_ref[...].astype(jnp.bfloat16)
    h = jnp.dot(x, w1s[...], preferred_element_type=jnp.float32)
    h = jnp.maximum(h + t1, 0.0).astype(jnp.bfloat16)
    h = jnp.dot(h, w2s[...], preferred_element_type=jnp.float32)
    h = jnp.maximum(h + t2, 0.0).astype(jnp.bfloat16)
    o_ref[...] = (jnp.dot(h, w3s[...], preferred_element_type=jnp.float32)
                  + b3)


def kernel(x, w1, b1, g1, be1, m1, v1, w2, b2, g2, be2, m2, v2, w3, b3):
    B, dim_in = x.shape
    l = w1.shape[1]
    dim_out = w3.shape[1]
    dim_out_p = max(128, _round_up(dim_out, 128))
    if dim_out_p != dim_out:
        w3 = jnp.pad(w3, ((0, 0), (0, dim_out_p - dim_out)))
        b3 = jnp.pad(b3, ((0, 0), (0, dim_out_p - dim_out)))

    s1 = g1 * lax.rsqrt(v1 + _EPS)
    t1 = (b1 - m1) * s1 + be1
    s2 = g2 * lax.rsqrt(v2 + _EPS)
    t2 = (b2 - m2) * s2 + be2
    lv = max(l, dim_out_p)
    pad_l = lambda a: jnp.pad(a, ((0, 0), (0, lv - a.shape[1])))
    vcat = jnp.concatenate(
        [pad_l(s1), pad_l(t1), pad_l(s2), pad_l(t2), pad_l(b3)], axis=0)

    TB = 1024 if B >= 1024 else max(8, _round_up(B, 8))
    B_pad = _round_up(B, TB)
    if B_pad != B:
        x = jnp.pad(x, ((0, B_pad - B), (0, 0)))
    grid = (B_pad // TB,)

    bf2, f4 = 2, 4
    footprint = ((f4 + bf2) * (dim_in * l + l * l + l * dim_out_p)
                 + f4 * 5 * lv
                 + 2 * (f4 * TB * dim_in + f4 * TB * dim_out_p)
                 + f4 * TB * l + bf2 * TB * l)
    vmem_limit = int(min(max(2 * footprint, 16 << 20), 60 << 20))

    const = lambda shape: pl.BlockSpec(shape, lambda i: (0, 0))
    body = functools.partial(_mlp3_body, dim_out_p=dim_out_p)
    out_p = pl.pallas_call(
        body,
        out_shape=jax.ShapeDtypeStruct((B_pad, dim_out_p), jnp.float32),
        grid=grid,
        in_specs=[
            pl.BlockSpec((TB, dim_in), lambda i: (i, 0)),
            const(w1.shape), const(w2.shape), const(w3.shape),
            const(vcat.shape),
        ],
        out_specs=pl.BlockSpec((TB, dim_out_p), lambda i: (i, 0)),
        scratch_shapes=[
            pltpu.VMEM((dim_in, l), jnp.bfloat16),
            pltpu.VMEM((l, l), jnp.bfloat16),
            pltpu.VMEM((l, dim_out_p), jnp.bfloat16),
        ],
        compiler_params=pltpu.CompilerParams(
            dimension_semantics=("parallel",),
            vmem_limit_bytes=vmem_limit,
        ),
    )(x, w1, w2, w3, vcat)

    return out_p[:B, :dim_out]
